# baseline (speedup 1.0000x reference)
import functools

import numpy as np
import jax
import jax.numpy as jnp
from jax.experimental import pallas as pl
from jax.experimental.pallas import tpu as pltpu

CTX = 2048
TOPK = 32
NKEYS = 512
D2 = 512
ROWS_PER_BLOCK = 512
NEG_INF = float("-inf")

_pairs = [(i, j) for i in range(TOPK) for j in range(TOPK)
          if (i + 1) * (j + 1) <= TOPK]
NCAND = 128
assert len(_pairs) <= NCAND
_ci = np.array([p[0] for p in _pairs])
_cj = np.array([p[1] for p in _pairs])
_A_np = np.zeros((TOPK, NCAND), np.float32)
_B_np = np.zeros((TOPK, NCAND), np.float32)
_A_np[_ci, np.arange(len(_pairs))] = 1.0
_B_np[_cj, np.arange(len(_pairs))] = 1.0
_C_np = np.full((1, NCAND), -np.inf, np.float32)
_C_np[0, : len(_pairs)] = 0.0

def _dot(a, b, precision=jax.lax.Precision.HIGHEST):
    return jax.lax.dot_general(
        a, b, (((1,), (0,)), ((), ())),
        precision=precision, preferred_element_type=jnp.float32)


def _topk_desc(v, k, payload=None):
    t, n = v.shape
    lane = jax.lax.broadcasted_iota(jnp.int32, (1, n), 1)
    kiota = jax.lax.broadcasted_iota(jnp.int32, (1, k), 1)
    big = jnp.int32(np.int32(2**30))
    acc_v = jnp.full((t, k), NEG_INF, dtype=jnp.float32)
    acc_i = jnp.zeros((t, k), dtype=jnp.int32)
    acc_p = None if payload is None else jnp.zeros((t, k), dtype=payload.dtype)
    for ki in range(k):
        m = jnp.max(v, axis=-1, keepdims=True)
        cand = jnp.where(v == m, lane, big)
        am = jnp.min(cand, axis=-1, keepdims=True)
        sel = cand == am
        acc_v = jnp.where(kiota == ki, m, acc_v)
        acc_i = jnp.where(kiota == ki, am, acc_i)
        if payload is not None:
            p = jnp.max(jnp.where(sel, payload, jnp.int32(-1)),
                        axis=-1, keepdims=True)
            acc_p = jnp.where(kiota == ki, p, acc_p)
        v = jnp.where(sel, NEG_INF, v)
    if payload is None:
        return acc_v, acc_i
    return acc_v, acc_i, acc_p


SR = 32


def _pkm_kernel(x_ref, k0_ref, k1_ref, w_ref, b_ref, a_ref, bb_ref, c_ref,
                out_s_ref, out_i_ref,
                d0_ref, d1_ref, sx_ref, sy_ref, ixf_ref, iyf_ref,
                cs_ref, ci_ref):
    xb = x_ref[...]
    w = w_ref[...]
    b = b_ref[...]
    eps = jnp.float32(1e-5)
    t = xb.shape[0]

    def ln(h):
        mu = jnp.mean(h, axis=-1, keepdims=True)
        hc = h - mu
        var = jnp.mean(hc * hc, axis=-1, keepdims=True)
        return hc / jnp.sqrt(var + eps) * w + b

    d0_ref[...] = _dot(ln(xb[:, :D2]), k0_ref[...],
                       precision=jax.lax.Precision.DEFAULT)
    d1_ref[...] = _dot(ln(xb[:, D2:]), k1_ref[...],
                       precision=jax.lax.Precision.DEFAULT)

    def strip1(s, carry):
        r = pl.multiple_of(s * SR, SR)
        sxv, sxi = _topk_desc(d0_ref[pl.ds(r, SR), :], TOPK)
        syv, syi = _topk_desc(d1_ref[pl.ds(r, SR), :], TOPK)
        sx_ref[pl.ds(r, SR), :] = sxv
        sy_ref[pl.ds(r, SR), :] = syv
        ixf_ref[pl.ds(r, SR), :] = sxi.astype(jnp.float32)
        iyf_ref[pl.ds(r, SR), :] = syi.astype(jnp.float32)
        return carry

    jax.lax.fori_loop(0, t // SR, strip1, 0, unroll=False)

    A = a_ref[...]
    B = bb_ref[...]
    cs_ref[...] = _dot(sx_ref[...], A) + _dot(sy_ref[...], B) + c_ref[...]
    ci_ref[...] = (_dot(ixf_ref[...] * np.float32(NKEYS), A)
                   + _dot(iyf_ref[...], B))

    def strip2(s, carry):
        r = pl.multiple_of(s * SR, SR)
        cs = cs_ref[pl.ds(r, SR), :]
        ci = ci_ref[pl.ds(r, SR), :].astype(jnp.int32)
        fin_s, _, fin_i = _topk_desc(cs, TOPK, payload=ci)
        out_s_ref[pl.ds(r, SR), :] = fin_s
        out_i_ref[pl.ds(r, SR), :] = fin_i
        return carry

    jax.lax.fori_loop(0, t // SR, strip2, 0, unroll=False)


@jax.jit
def kernel(x, keys, norm_w, norm_b):
    rows = x.shape[0]
    t = ROWS_PER_BLOCK
    grid = (rows // t,)
    k0 = keys[:, 0, :].T
    k1 = keys[:, 1, :].T
    w2 = norm_w.reshape(1, D2)
    b2 = norm_b.reshape(1, D2)
    A = jnp.asarray(_A_np)
    B = jnp.asarray(_B_np)
    C = jnp.asarray(_C_np)

    const = lambda shape: pl.BlockSpec(shape, lambda i: (0, 0))
    out_s, out_i = pl.pallas_call(
        _pkm_kernel,
        grid=grid,
        in_specs=[
            pl.BlockSpec((t, 2 * D2), lambda i: (i, 0)),
            const((D2, NKEYS)),
            const((D2, NKEYS)),
            const((1, D2)),
            const((1, D2)),
            const((TOPK, NCAND)),
            const((TOPK, NCAND)),
            const((1, NCAND)),
        ],
        out_specs=[
            pl.BlockSpec((t, TOPK), lambda i: (i, 0)),
            pl.BlockSpec((t, TOPK), lambda i: (i, 0)),
        ],
        out_shape=[
            jax.ShapeDtypeStruct((rows, TOPK), jnp.float32),
            jax.ShapeDtypeStruct((rows, TOPK), jnp.int32),
        ],
        scratch_shapes=[
            pltpu.VMEM((t, NKEYS), jnp.float32),
            pltpu.VMEM((t, NKEYS), jnp.float32),
            pltpu.VMEM((t, TOPK), jnp.float32),
            pltpu.VMEM((t, TOPK), jnp.float32),
            pltpu.VMEM((t, TOPK), jnp.float32),
            pltpu.VMEM((t, TOPK), jnp.float32),
            pltpu.VMEM((t, NCAND), jnp.float32),
            pltpu.VMEM((t, NCAND), jnp.float32),
        ],
    )(x, k0, k1, w2, b2, A, B, C)
    return (out_s, out_i)


# speedup vs baseline: 2.6800x; 2.6800x over previous
import functools

import numpy as np
import jax
import jax.numpy as jnp
from jax.experimental import pallas as pl
from jax.experimental.pallas import tpu as pltpu

CTX = 2048
TOPK = 32
NKEYS = 512
D2 = 512
ROWS_PER_BLOCK = 512
NEG_INF = float("-inf")

_pairs = [(i, j) for i in range(TOPK) for j in range(TOPK)
          if (i + 1) * (j + 1) <= TOPK]
NCAND = 128
assert len(_pairs) <= NCAND
_ci = np.array([p[0] for p in _pairs])
_cj = np.array([p[1] for p in _pairs])
_A_np = np.zeros((TOPK, NCAND), np.float32)
_B_np = np.zeros((TOPK, NCAND), np.float32)
_A_np[_ci, np.arange(len(_pairs))] = 1.0
_B_np[_cj, np.arange(len(_pairs))] = 1.0
_C_np = np.full((1, NCAND), -np.inf, np.float32)
_C_np[0, : len(_pairs)] = 0.0


def _dot(a, b, precision=jax.lax.Precision.HIGHEST):
    return jax.lax.dot_general(
        a, b, (((1,), (0,)), ((), ())),
        precision=precision, preferred_element_type=jnp.float32)


def _topk_desc(v, k, payload=None):
    t, n = v.shape
    lane = jax.lax.broadcasted_iota(jnp.int32, (1, n), 1)
    kiota = jax.lax.broadcasted_iota(jnp.int32, (1, k), 1)
    acc_v = jnp.full((t, k), NEG_INF, dtype=jnp.float32)
    acc_i = jnp.zeros((t, k), dtype=jnp.int32)
    acc_p = None if payload is None else jnp.zeros((t, k), dtype=payload.dtype)
    for ki in range(k):
        m = jnp.max(v, axis=-1, keepdims=True)
        am = jnp.argmax(v, axis=-1, keepdims=True).astype(jnp.int32)
        sel = lane == am
        acc_v = jnp.where(kiota == ki, m, acc_v)
        acc_i = jnp.where(kiota == ki, am, acc_i)
        if payload is not None:
            p = jnp.max(jnp.where(sel, payload, jnp.int32(-1)),
                        axis=-1, keepdims=True)
            acc_p = jnp.where(kiota == ki, p, acc_p)
        v = jnp.where(sel, NEG_INF, v)
    if payload is None:
        return acc_v, acc_i
    return acc_v, acc_i, acc_p


def _pkm_kernel(x_ref, k0_ref, k1_ref, w_ref, b_ref, a_ref, bb_ref, c_ref,
                out_s_ref, out_i_ref):
    xb = x_ref[...]
    w = w_ref[...]
    b = b_ref[...]
    eps = jnp.float32(1e-5)

    def ln(h):
        mu = jnp.mean(h, axis=-1, keepdims=True)
        hc = h - mu
        var = jnp.mean(hc * hc, axis=-1, keepdims=True)
        return hc / jnp.sqrt(var + eps) * w + b

    q0 = ln(xb[:, :D2])
    q1 = ln(xb[:, D2:])
    dots0 = _dot(q0, k0_ref[...], precision=jax.lax.Precision.DEFAULT)
    dots1 = _dot(q1, k1_ref[...], precision=jax.lax.Precision.DEFAULT)

    sx, ix = _topk_desc(dots0, TOPK)
    sy, iy = _topk_desc(dots1, TOPK)

    A = a_ref[...]
    B = bb_ref[...]
    cand_s = _dot(sx, A) + _dot(sy, B) + c_ref[...]
    cand_if = (_dot(ix.astype(jnp.float32) * np.float32(NKEYS), A)
               + _dot(iy.astype(jnp.float32), B))
    cand_idx = cand_if.astype(jnp.int32)

    fin_s, _, fin_i = _topk_desc(cand_s, TOPK, payload=cand_idx)
    out_s_ref[...] = fin_s
    out_i_ref[...] = fin_i


@jax.jit
def kernel(x, keys, norm_w, norm_b):
    rows = x.shape[0]
    t = ROWS_PER_BLOCK
    grid = (rows // t,)
    k0 = keys[:, 0, :].T
    k1 = keys[:, 1, :].T
    w2 = norm_w.reshape(1, D2)
    b2 = norm_b.reshape(1, D2)
    A = jnp.asarray(_A_np)
    B = jnp.asarray(_B_np)
    C = jnp.asarray(_C_np)

    const = lambda shape: pl.BlockSpec(shape, lambda i: (0, 0))
    out_s, out_i = pl.pallas_call(
        _pkm_kernel,
        grid=grid,
        in_specs=[
            pl.BlockSpec((t, 2 * D2), lambda i: (i, 0)),
            const((D2, NKEYS)),
            const((D2, NKEYS)),
            const((1, D2)),
            const((1, D2)),
            const((TOPK, NCAND)),
            const((TOPK, NCAND)),
            const((1, NCAND)),
        ],
        out_specs=[
            pl.BlockSpec((t, TOPK), lambda i: (i, 0)),
            pl.BlockSpec((t, TOPK), lambda i: (i, 0)),
        ],
        out_shape=[
            jax.ShapeDtypeStruct((rows, TOPK), jnp.float32),
            jax.ShapeDtypeStruct((rows, TOPK), jnp.int32),
        ],
    )(x, k0, k1, w2, b2, A, B, C)
    return (out_s, out_i)


# speedup vs baseline: 4.9725x; 1.8554x over previous
import functools

import numpy as np
import jax
import jax.numpy as jnp
from jax.experimental import pallas as pl
from jax.experimental.pallas import tpu as pltpu

CTX = 2048
TOPK = 32
NKEYS = 512
D2 = 512
ROWS_PER_BLOCK = 512
NEG_INF = float("-inf")

_pairs = [(i, j) for i in range(TOPK) for j in range(TOPK)
          if (i + 1) * (j + 1) <= TOPK]
NCAND = 128
assert len(_pairs) <= NCAND
_ci = np.array([p[0] for p in _pairs])
_cj = np.array([p[1] for p in _pairs])
_A_np = np.zeros((TOPK, NCAND), np.float32)
_B_np = np.zeros((TOPK, NCAND), np.float32)
_A_np[_ci, np.arange(len(_pairs))] = 1.0
_B_np[_cj, np.arange(len(_pairs))] = 1.0
_A512_np = _A_np * np.float32(NKEYS)
_C_np = np.full((1, NCAND), -np.inf, np.float32)
_C_np[0, : len(_pairs)] = 0.0


def _dot(a, b, precision=jax.lax.Precision.HIGHEST):
    return jax.lax.dot_general(
        a, b, (((1,), (0,)), ((), ())),
        precision=precision, preferred_element_type=jnp.float32)


def _topk_desc(v, k, payload=None, want_idx=True):
    t, n = v.shape
    lane = jax.lax.broadcasted_iota(jnp.int32, (1, n), 1)
    kiota = jax.lax.broadcasted_iota(jnp.int32, (1, k), 1)
    big = jnp.int32(np.int32(2**30))
    acc_v = jnp.full((t, k), NEG_INF, dtype=jnp.float32)
    acc_i = jnp.zeros((t, k), dtype=jnp.int32) if want_idx else None
    acc_p = None if payload is None else jnp.zeros((t, k), dtype=payload.dtype)
    for ki in range(k):
        m = jnp.max(v, axis=-1, keepdims=True)
        cand = jnp.where(v == m, lane, big)
        am = jnp.min(cand, axis=-1, keepdims=True)
        sel = cand == am
        acc_v = jnp.where(kiota == ki, m, acc_v)
        if want_idx:
            acc_i = jnp.where(kiota == ki, am, acc_i)
        if payload is not None:
            p = jnp.max(jnp.where(sel, payload, jnp.int32(-1)),
                        axis=-1, keepdims=True)
            acc_p = jnp.where(kiota == ki, p, acc_p)
        v = jnp.where(sel, NEG_INF, v)
    return acc_v, acc_i, acc_p


def _pkm_kernel(x_ref, k0_ref, k1_ref, w_ref, b_ref, a_ref, bb_ref,
                a512_ref, c_ref, out_s_ref, out_i_ref):
    xb = x_ref[...]
    w = w_ref[...]
    b = b_ref[...]
    eps = jnp.float32(1e-5)

    def ln(h):
        mu = jnp.mean(h, axis=-1, keepdims=True)
        hc = h - mu
        var = jnp.mean(hc * hc, axis=-1, keepdims=True)
        return hc / jnp.sqrt(var + eps) * w + b

    q0 = ln(xb[:, :D2])
    q1 = ln(xb[:, D2:])
    dots0 = _dot(q0, k0_ref[...], precision=jax.lax.Precision.DEFAULT)
    dots1 = _dot(q1, k1_ref[...], precision=jax.lax.Precision.DEFAULT)

    t = xb.shape[0]
    d_all = jnp.concatenate([dots0, dots1], axis=0)
    s_all, i_all, _ = _topk_desc(d_all, TOPK)
    sx, ix = s_all[:t], i_all[:t]
    sy, iy = s_all[t:], i_all[t:]

    A = a_ref[...]
    B = bb_ref[...]
    cand_s = _dot(sx, A) + _dot(sy, B) + c_ref[...]
    cand_if = _dot(ix.astype(jnp.float32), a512_ref[...]) + _dot(
        iy.astype(jnp.float32), B)
    cand_idx = cand_if.astype(jnp.int32)

    fin_s, _, fin_i = _topk_desc(cand_s, TOPK, payload=cand_idx,
                                 want_idx=False)
    out_s_ref[...] = fin_s
    out_i_ref[...] = fin_i


@jax.jit
def kernel(x, keys, norm_w, norm_b):
    rows = x.shape[0]
    t = ROWS_PER_BLOCK
    grid = (rows // t,)
    k0 = keys[:, 0, :].T
    k1 = keys[:, 1, :].T
    w2 = norm_w.reshape(1, D2)
    b2 = norm_b.reshape(1, D2)
    A = jnp.asarray(_A_np)
    B = jnp.asarray(_B_np)
    A512 = jnp.asarray(_A512_np)
    C = jnp.asarray(_C_np)

    const = lambda shape: pl.BlockSpec(shape, lambda i: (0, 0))
    out_s, out_i = pl.pallas_call(
        _pkm_kernel,
        grid=grid,
        in_specs=[
            pl.BlockSpec((t, 2 * D2), lambda i: (i, 0)),
            const((D2, NKEYS)),
            const((D2, NKEYS)),
            const((1, D2)),
            const((1, D2)),
            const((TOPK, NCAND)),
            const((TOPK, NCAND)),
            const((TOPK, NCAND)),
            const((1, NCAND)),
        ],
        out_specs=[
            pl.BlockSpec((t, TOPK), lambda i: (i, 0)),
            pl.BlockSpec((t, TOPK), lambda i: (i, 0)),
        ],
        out_shape=[
            jax.ShapeDtypeStruct((rows, TOPK), jnp.float32),
            jax.ShapeDtypeStruct((rows, TOPK), jnp.int32),
        ],
    )(x, k0, k1, w2, b2, A, B, A512, C)
    return (out_s, out_i)
